# baseline (speedup 1.0000x reference)
import functools

import jax
import jax.numpy as jnp
from jax import lax
from jax.experimental import pallas as pl
from jax.experimental.pallas import tpu as pltpu
from jax.experimental.pallas import tpu_sc as plsc

E = 800000
D = 64
W = 2 * D
NUM_CORES = 2
NUM_SUBCORES = 16
NW = NUM_CORES * NUM_SUBCORES
CP = 320
T = (E // 2) // CP
Q, R = divmod(T, NW)
MAXT = Q + 1
GROUPS = MAXT * CP // 16
BUF = CP * W


@jax.jit
def _sc_embed(idx, ptab_flat):
    mesh = plsc.VectorSubcoreMesh(core_axis_name="c", subcore_axis_name="s")

    @functools.partial(
        pl.kernel,
        mesh=mesh,
        out_type=jax.ShapeDtypeStruct((E * D,), jnp.float32),
        scratch_types=[
            pltpu.VMEM((MAXT * 2 * CP,), jnp.int32),
            pltpu.VMEM((MAXT * CP,), jnp.int32),
            pltpu.VMEM((16 * W,), jnp.float32),
            pltpu.VMEM((2 * BUF,), jnp.float32),
            pltpu.SemaphoreType.DMA,
        ],
        compiler_params=pltpu.CompilerParams(needs_layout_passes=False),
    )
    def k(idx_hbm, ptab_hbm, out_hbm, idx_v, pair_v, ptab_v, rows_v, wsem):
        wid = lax.axis_index("s") * NUM_CORES + lax.axis_index("c")
        start = jnp.minimum(wid * Q + jnp.minimum(wid, R), T - MAXT)

        pltpu.sync_copy(ptab_hbm, ptab_v)
        pltpu.sync_copy(idx_hbm.at[pl.ds(start * 2 * CP, MAXT * 2 * CP)], idx_v)

        iota = lax.iota(jnp.int32, 16)
        two_iota = iota * 2
        iota_w = iota * W

        def pair_body(g, carry):
            pos = two_iota + g * 32
            ev = plsc.load_gather(idx_v, [pos])
            od = plsc.load_gather(idx_v, [pos + 1])
            pair_v[pl.ds(g * 16, 16)] = jnp.bitwise_and(ev * 4 + od, 15)
            return carry

        lax.fori_loop(0, GROUPS, pair_body, 0)

        def out_slice(ci):
            return out_hbm.at[pl.ds((start + ci) * BUF, BUF)]

        def buf_slice(boff):
            return rows_v.at[pl.ds(boff, BUF)]

        def xfer_body(ci, carry):
            boff = jnp.bitwise_and(ci, 1) * BUF

            @pl.when(ci >= 2)
            def _():
                pltpu.make_async_copy(buf_slice(boff), out_slice(ci - 2), wsem).wait()

            def grp(g, c2):
                pairvec = pair_v[pl.ds(ci * CP + g * 16, 16)]
                src0 = pairvec * W
                dst0 = iota_w + boff + g * (16 * W)
                for j in range(W):
                    v = plsc.load_gather(ptab_v, [src0 + j])
                    plsc.store_scatter(rows_v, [dst0 + j], v)
                return c2

            lax.fori_loop(0, CP // 16, grp, 0)
            pltpu.async_copy(buf_slice(boff), out_slice(ci), wsem)
            return carry

        lax.fori_loop(0, MAXT, xfer_body, 0)

        for ci in (MAXT - 2, MAXT - 1):
            boff = (ci & 1) * BUF
            pltpu.make_async_copy(buf_slice(boff), out_slice(ci), wsem).wait()

    return k(idx, ptab_flat)


def kernel(edge_type, table):
    idx = edge_type.astype(jnp.int32)
    ptab = jnp.concatenate(
        [jnp.repeat(table, 4, axis=0), jnp.tile(table, (4, 1))], axis=1
    )
    out = _sc_embed(idx, ptab.reshape(-1))
    return out.reshape(E, D)


# speedup vs baseline: 3.5734x; 3.5734x over previous
import functools

import jax
import jax.numpy as jnp
from jax import lax
from jax.experimental import pallas as pl
from jax.experimental.pallas import tpu as pltpu
from jax.experimental.pallas import tpu_sc as plsc

E = 800000
D = 64
NUM_CORES = 2
NUM_SUBCORES = 16
NW = NUM_CORES * NUM_SUBCORES
CP = 320
T = (E // 2) // CP
Q, R = divmod(T, NW)
MAXT = Q + 1
GROUPS = MAXT * CP // 16
NREP = 256
NSPLIT = 4


@jax.jit
def _sc_embed(idx, ptab):
    mesh = plsc.VectorSubcoreMesh(core_axis_name="c", subcore_axis_name="s")

    @functools.partial(
        pl.kernel,
        mesh=mesh,
        out_type=jax.ShapeDtypeStruct((E // 2, 2 * D), jnp.float32),
        scratch_types=[
            pltpu.VMEM((MAXT * 2 * CP,), jnp.int32),
            pltpu.VMEM((MAXT * CP,), jnp.int32),
            pltpu.VMEM((2 * CP, 2 * D), jnp.float32),
            [pltpu.SemaphoreType.DMA] * (2 * NSPLIT),
            pltpu.SemaphoreType.DMA,
            pltpu.SemaphoreType.DMA,
        ],
        compiler_params=pltpu.CompilerParams(needs_layout_passes=False),
    )
    def k(idx_hbm, ptab_hbm, out_hbm, idx_v, pair_v, rows_v, gsems, w0, w1):
        wid = lax.axis_index("s") * NUM_CORES + lax.axis_index("c")
        start = jnp.minimum(wid * Q + jnp.minimum(wid, R), T - MAXT)

        pltpu.sync_copy(idx_hbm.at[pl.ds(start * 2 * CP, MAXT * 2 * CP)], idx_v)

        two_iota = lax.iota(jnp.int32, 16) * 2
        half_iota = lax.iota(jnp.int32, 16)

        def pair_body(g, carry):
            pos = two_iota + g * 32
            ev = plsc.load_gather(idx_v, [pos])
            od = plsc.load_gather(idx_v, [pos + 1])
            rep = jnp.bitwise_and((wid * GROUPS + g) * 16 + half_iota, NREP - 1)
            pair_v[pl.ds(g * 16, 16)] = (
                jnp.bitwise_and(ev * 4 + od, 15) + rep * 16
            )
            return carry

        lax.fori_loop(0, GROUPS, pair_body, 0)

        wsem = (w0, w1)
        SP = CP // NSPLIT

        def gather(ci, b):
            descs = [
                pltpu.async_copy(
                    ptab_hbm.at[pair_v.at[pl.ds(ci * CP + q * SP, SP)]],
                    rows_v.at[pl.ds(b * CP + q * SP, SP)],
                    gsems[b * NSPLIT + q],
                )
                for q in range(NSPLIT)
            ]

            class _Multi:
                def wait(self):
                    for d in descs:
                        d.wait()

            return _Multi()

        def write(ci, b):
            return pltpu.async_copy(
                rows_v.at[pl.ds(b * CP, CP)],
                out_hbm.at[pl.ds((start + ci) * CP, CP)],
                wsem[b],
            )

        g_desc = [gather(0, 0), None]
        w_desc = [None, None]
        for ci in range(MAXT):
            b = ci & 1
            g_desc[b].wait()
            if ci + 1 < MAXT:
                ob = 1 - b
                if w_desc[ob] is not None:
                    w_desc[ob].wait()
                g_desc[ob] = gather(ci + 1, ob)
            w_desc[b] = write(ci, b)
        w_desc[(MAXT - 1) & 1].wait()
        w_desc[(MAXT - 2) & 1].wait()

    return k(idx, ptab)


def kernel(edge_type, table):
    idx = edge_type.astype(jnp.int32)
    ptab = jnp.concatenate(
        [jnp.repeat(table, 4, axis=0), jnp.tile(table, (4, 1))], axis=1
    )
    ptab = jnp.tile(ptab, (NREP, 1))
    out2 = _sc_embed(idx, ptab)
    return out2.reshape(E, D)
